# baseline (speedup 1.0000x reference)
import functools
import jax
import jax.numpy as jnp
from jax import lax
from jax.experimental import pallas as pl
from jax.experimental.pallas import tpu as pltpu
from jax.experimental.pallas import tpu_sc as plsc

BATCH = 1024
SEQ = 200
DIM = 128
NW = 32
DOCS_PER_W = BATCH // NW
G = 2
NGROUPS = DOCS_PER_W // G

_mesh = plsc.VectorSubcoreMesh(core_axis_name="c", subcore_axis_name="s")


@functools.partial(
    pl.kernel,
    out_type=jax.ShapeDtypeStruct((BATCH, SEQ, DIM), jnp.float32),
    mesh=_mesh,
    scratch_types=[
        pltpu.VMEM((SEQ, DIM), jnp.float32),
        pltpu.VMEM((2, G, SEQ, DIM), jnp.float32),
        pltpu.SemaphoreType.DMA,
        pltpu.SemaphoreType.DMA,
        pltpu.SemaphoreType.DMA,
        pltpu.SemaphoreType.DMA,
    ],
)
def _sc_add(x_hbm, pe_hbm, out_hbm, pe_v, buf_v, ls0, ls1, ss0, ss1):
    wid = lax.axis_index("s") * 2 + lax.axis_index("c")
    base = wid * DOCS_PER_W
    lsem = (ls0, ls1)
    ssem = (ss0, ss1)

    def start_load(g, slot):
        return pltpu.async_copy(
            x_hbm.at[pl.ds(base + g * G, G)], buf_v.at[slot], lsem[slot])

    def start_store(g, slot):
        return pltpu.async_copy(
            buf_v.at[slot], out_hbm.at[pl.ds(base + g * G, G)], ssem[slot])

    load_h = [start_load(0, 0), start_load(1, 1)]
    pltpu.sync_copy(pe_hbm, pe_v)

    store_h = [None, None]
    for g in range(NGROUPS):
        s = g & 1
        load_h[s].wait()

        @plsc.parallel_loop(0, SEQ, 1, unroll=2)
        def _body(r):
            for c in range(DIM // 16):
                v = pe_v[r, pl.ds(c * 16, 16)]
                for d in range(G):
                    buf_v[s, d, r, pl.ds(c * 16, 16)] += v

        store_h[s] = start_store(g, s)
        if g + 2 < NGROUPS:
            store_h[s].wait()
            load_h[s] = start_load(g + 2, s)
    store_h[0].wait()
    store_h[1].wait()


def kernel(padded_encoded_input_docs, pos_emb):
    return _sc_add(padded_encoded_input_docs, pos_emb)


# speedup vs baseline: 1.1287x; 1.1287x over previous
import functools
import jax
import jax.numpy as jnp
from jax import lax
from jax.experimental import pallas as pl
from jax.experimental.pallas import tpu as pltpu
from jax.experimental.pallas import tpu_sc as plsc

BATCH = 1024
SEQ = 200
DIM = 128
NW = 32
SC_DOCS = 128
TC_DOCS = BATCH - SC_DOCS
DOCS_PER_W = SC_DOCS // NW
G = 2
NGROUPS = DOCS_PER_W // G
BLOCK_B = 128

_mesh = plsc.VectorSubcoreMesh(core_axis_name="c", subcore_axis_name="s")


@functools.partial(
    pl.kernel,
    out_type=jax.ShapeDtypeStruct((SC_DOCS, SEQ, DIM), jnp.float32),
    mesh=_mesh,
    cost_estimate=pl.CostEstimate(
        flops=2 * SC_DOCS * SEQ * DIM,
        bytes_accessed=2 * SC_DOCS * SEQ * DIM * 4,
        transcendentals=0,
    ),
    scratch_types=[
        pltpu.VMEM((SEQ, DIM), jnp.float32),
        pltpu.VMEM((2, G, SEQ, DIM), jnp.float32),
        pltpu.SemaphoreType.DMA,
        pltpu.SemaphoreType.DMA,
        pltpu.SemaphoreType.DMA,
        pltpu.SemaphoreType.DMA,
    ],
)
def _sc_add(x_hbm, pe_hbm, out_hbm, pe_v, buf_v, ls0, ls1, ss0, ss1):
    wid = lax.axis_index("s") * 2 + lax.axis_index("c")
    base = wid * DOCS_PER_W
    lsem = (ls0, ls1)
    ssem = (ss0, ss1)

    pltpu.sync_copy(pe_hbm, pe_v)

    def start_load(g, slot):
        return pltpu.async_copy(
            x_hbm.at[pl.ds(base + g * G, G)], buf_v.at[slot], lsem[slot])

    def start_store(g, slot):
        return pltpu.async_copy(
            buf_v.at[slot], out_hbm.at[pl.ds(base + g * G, G)], ssem[slot])

    load_h = [start_load(0, 0), None]
    store_h = [None, None]
    for g in range(NGROUPS):
        s = g & 1
        load_h[s].wait()
        if g + 1 < NGROUPS:
            if store_h[1 - s] is not None:
                store_h[1 - s].wait()
            load_h[1 - s] = start_load(g + 1, 1 - s)

        @plsc.parallel_loop(0, SEQ, 1, unroll=2)
        def _body(r):
            for c in range(DIM // 16):
                v = pe_v[r, pl.ds(c * 16, 16)]
                for d in range(G):
                    buf_v[s, d, r, pl.ds(c * 16, 16)] += v

        store_h[s] = start_store(g, s)
    store_h[0].wait()
    store_h[1].wait()


def _tc_body(x_ref, pe_ref, o_ref):
    o_ref[...] = x_ref[...] + pe_ref[...][None, :, :]


def _tc_add(x, pos_emb):
    grid = (TC_DOCS // BLOCK_B,)
    return pl.pallas_call(
        _tc_body,
        grid=grid,
        in_specs=[
            pl.BlockSpec((BLOCK_B, SEQ, DIM),
                         lambda i: (i + SC_DOCS // BLOCK_B, 0, 0)),
            pl.BlockSpec((SEQ, DIM), lambda i: (0, 0)),
        ],
        out_specs=pl.BlockSpec((BLOCK_B, SEQ, DIM),
                               lambda i: (i + SC_DOCS // BLOCK_B, 0, 0)),
        out_shape=jax.ShapeDtypeStruct((BATCH, SEQ, DIM), jnp.float32),
    )(x, pos_emb)


def kernel(padded_encoded_input_docs, pos_emb):
    x = padded_encoded_input_docs
    sc_part = _sc_add(x, pos_emb)
    tc_full = _tc_add(x, pos_emb)
    return lax.dynamic_update_slice(tc_full, sc_part, (0, 0, 0))
